# baseline (speedup 1.0000x reference)
import functools

import jax
import jax.numpy as jnp
from jax import lax
from jax.experimental import pallas as pl
from jax.experimental.pallas import tpu as pltpu
from jax.experimental.pallas import tpu_sc as plsc

_LANES = 16


def _make_sc_embed(B, L, V, D):
    NW = 32
    assert B % NW == 0
    bpw = B // NW
    nd = D // _LANES
    cA, cB = 104, L - 104
    RA = 112
    RT = RA + cB
    assert cA % 8 == 0 and cA <= 128 and 0 < cB <= 128 and cB % _LANES == 0

    mesh = plsc.VectorSubcoreMesh(core_axis_name="c", subcore_axis_name="s")

    @functools.partial(
        pl.kernel,
        mesh=mesh,
        compiler_params=pltpu.CompilerParams(use_tc_tiling_on_sc=False),
        out_type=jax.ShapeDtypeStruct((B, L, D), jnp.float32),
        scratch_types=[
            pltpu.VMEM((2, RA), jnp.int32),
            pltpu.VMEM((2, RA), jnp.float32),
            pltpu.VMEM((RT, D), jnp.float32),
            pltpu.VMEM((1, D), jnp.float32),
            pltpu.VMEM((RT, D), jnp.float32),
            pltpu.SemaphoreType.DMA,
        ],
    )
    def sc_embed(idx_hbm, mask_hbm, table_hbm, me_hbm, pos_hbm, out_hbm,
                 idx_v, mask_v, pos_v, me_v, buf, sem):
        wid = lax.axis_index("s") * 2 + lax.axis_index("c")
        pltpu.sync_copy(pos_hbm.at[pl.ds(0, cA)], pos_v.at[pl.ds(0, cA)])
        pltpu.sync_copy(pos_hbm.at[pl.ds(cA, cB)], pos_v.at[pl.ds(RA, cB)])
        pltpu.sync_copy(me_hbm, me_v)
        emb = [me_v[0, pl.ds(_LANES * j, _LANES)] for j in range(nd)]

        def chunk_body(g, _):
            bb = wid * bpw + g
            pltpu.sync_copy(idx_hbm.at[bb], idx_v)
            pltpu.sync_copy(mask_hbm.at[bb], mask_v)
            cp0 = pltpu.async_copy(
                table_hbm.at[idx_v.at[0, pl.ds(0, cA)]],
                buf.at[pl.ds(0, cA)], sem)
            cp1 = pltpu.async_copy(
                table_hbm.at[idx_v.at[1, pl.ds(0, cB)]],
                buf.at[pl.ds(RA, cB)], sem)
            cp0.wait()
            cp1.wait()

            for h, ngrp in ((0, RA // _LANES), (1, cB // _LANES)):
                def grp_body(t, _, h=h):
                    base = t * _LANES
                    m16 = mask_v[h, pl.ds(base, _LANES)]
                    for r in range(_LANES):
                        mr = jnp.full((_LANES,), m16[r], jnp.float32)
                        l = h * RA + base + r
                        for j in range(nd):
                            sl = pl.ds(_LANES * j, _LANES)
                            g16 = buf[l, sl]
                            p16 = pos_v[l, sl]
                            buf[l, sl] = emb[j] + mr * (g16 - emb[j]) + p16
                    return 0

                lax.fori_loop(0, ngrp, grp_body, 0)

            pltpu.sync_copy(buf.at[pl.ds(0, cA)],
                            out_hbm.at[bb, pl.ds(0, cA)])
            pltpu.sync_copy(buf.at[pl.ds(RA, cB)],
                            out_hbm.at[bb, pl.ds(cA, cB)])
            return 0

        lax.fori_loop(0, bpw, chunk_body, 0)

    return sc_embed


def _pack_regions(x, cA, RA, dtype):
    B, L = x.shape
    cB = L - cA
    a = jnp.pad(x[:, :cA].astype(dtype), ((0, 0), (0, RA - cA)))
    b = jnp.pad(x[:, cA:].astype(dtype), ((0, 0), (0, RA - cB)))
    return jnp.stack([a, b], axis=1)


def kernel(item_id, token_mask, item_table, mask_emb, pos_emb):
    B, L = item_id.shape
    V, D = item_table.shape
    cA, RA = 104, 112
    idxp = _pack_regions(item_id, cA, RA, jnp.int32)
    maskp = _pack_regions(token_mask, cA, RA, jnp.float32)
    sc_embed = _make_sc_embed(B, L, V, D)
    return sc_embed(idxp, maskp, item_table, mask_emb, pos_emb)
